# baseline (speedup 1.0000x reference)
import functools

import jax
import jax.numpy as jnp
from jax import lax
from jax.experimental import pallas as pl
from jax.experimental.pallas import tpu as pltpu
from jax.experimental.pallas import tpu_sc as plsc

BATCH = 4
SEQ = 2048
EMBED = 64
TOTAL = BATCH * SEQ


def _sc_dims():
    try:
        info = plsc.get_sparse_core_info()
        return info.num_cores, info.num_subcores
    except Exception:
        return 2, 16


@functools.cache
def _build():
    nc, ns = _sc_dims()
    nw = nc * ns
    bpw = TOTAL // nw
    assert TOTAL % nw == 0 and SEQ % bpw == 0
    mesh = plsc.VectorSubcoreMesh(core_axis_name="c", subcore_axis_name="s")

    @functools.partial(
        pl.kernel,
        mesh=mesh,
        out_type=jax.ShapeDtypeStruct((TOTAL, EMBED), jnp.float32),
        scratch_types=[
            pltpu.VMEM((bpw,), jnp.int32),
            pltpu.VMEM((bpw, EMBED), jnp.float32),
            pltpu.VMEM((bpw, EMBED), jnp.float32),
            pltpu.SemaphoreType.DMA,
        ],
        compiler_params=pltpu.CompilerParams(use_tc_tiling_on_sc=False),
    )
    def emb(x_hbm, tok_hbm, pos_hbm, out_hbm, idx_v, tok_v, pos_v, sem):
        wid = lax.axis_index("s") * nc + lax.axis_index("c")
        base = wid * bpw
        pos0 = base % SEQ
        pltpu.sync_copy(x_hbm.at[pl.ds(base, bpw)], idx_v)
        gather = pltpu.async_copy(tok_hbm.at[idx_v], tok_v, sem)
        pltpu.sync_copy(pos_hbm.at[pl.ds(pos0, bpw)], pos_v)
        gather.wait()

        def row_add(r, carry):
            for c in range(0, EMBED, 16):
                tok_v[r, pl.ds(c, 16)] = (
                    tok_v[r, pl.ds(c, 16)] + pos_v[r, pl.ds(c, 16)]
                )
            return carry

        lax.fori_loop(0, bpw, row_add, 0, unroll=4)
        pltpu.sync_copy(tok_v, out_hbm.at[pl.ds(base, bpw)])

    return emb


def kernel(x, tok_table, pos_table):
    b, s = x.shape
    out = _build()(x.reshape(-1).astype(jnp.int32), tok_table, pos_table)
    return out.reshape(b, s, tok_table.shape[1])
